# baseline (speedup 1.0000x reference)
import functools

import jax
import jax.numpy as jnp
from jax import lax
from jax.experimental import pallas as pl
from jax.experimental.pallas import tpu as pltpu
from jax.experimental.pallas import tpu_sc as plsc

N_CLASS = 1000000
DROPOUT_PROB = 0.1


def _drop_mask(n: int):
    u = jax.random.uniform(jax.random.key(1234), (n,))
    return (u < DROPOUT_PROB).astype(jnp.int32)


def _transpose_kernel(in_ref, out_ref):
    out_ref[...] = in_ref[...].T


@functools.lru_cache
def _build_transpose(hidden: int, n_rows: int, blk: int):
    grid = (pl.cdiv(n_rows, blk),)
    return pl.pallas_call(
        _transpose_kernel,
        grid=grid,
        in_specs=[pl.BlockSpec((hidden, blk), lambda i: (0, i))],
        out_specs=pl.BlockSpec((blk, hidden), lambda i: (i, 0)),
        out_shape=jax.ShapeDtypeStruct((n_rows, hidden), jnp.float32),
    )


@functools.lru_cache
def _build_gather(batch: int, hidden: int, n_rows: int):
    info = plsc.get_sparse_core_info()
    nc, ns, lanes = info.num_cores, info.num_subcores, info.num_lanes
    nw = nc * ns
    assert batch % (8 * nw) == 0 and hidden % lanes == 0
    b_per_w = batch // nw
    mesh = plsc.VectorSubcoreMesh(core_axis_name="c", subcore_axis_name="s")

    @functools.partial(
        pl.kernel,
        mesh=mesh,
        out_type=jax.ShapeDtypeStruct((batch, hidden), jnp.float32),
        compiler_params=pltpu.CompilerParams(use_tc_tiling_on_sc=False),
        scratch_types=[
            pltpu.VMEM((b_per_w,), jnp.int32),
            pltpu.VMEM((b_per_w,), jnp.int32),
            pltpu.VMEM((b_per_w,), jnp.int32),
            pltpu.VMEM((b_per_w, hidden), jnp.float32),
            pltpu.SemaphoreType.DMA,
        ],
    )
    def emb(table_hbm, labels_hbm, mask_hbm, out_hbm, lab_v, msk_v, idx_v, rows_v, sem):
        wid = lax.axis_index("s") * nc + lax.axis_index("c")
        base = wid * b_per_w
        pltpu.sync_copy(labels_hbm.at[pl.ds(base, b_per_w)], lab_v)
        pltpu.sync_copy(mask_hbm.at[pl.ds(base, b_per_w)], msk_v)
        for i in range(b_per_w // lanes):
            s = pl.ds(i * lanes, lanes)
            idx_v[s] = jnp.where(msk_v[s] != 0, N_CLASS, lab_v[s])
        pltpu.async_copy(table_hbm.at[idx_v], rows_v, sem).wait()
        pltpu.sync_copy(rows_v, out_hbm.at[pl.ds(base, b_per_w)])

    return emb


def kernel(labels, table):
    batch = labels.shape[0]
    n_rows, hidden = table.shape
    mask = _drop_mask(batch)
    table_rm = _build_transpose(hidden, n_rows, 2048)(table.T)
    emb = _build_gather(batch, hidden, n_rows)
    return emb(table_rm, labels.astype(jnp.int32), mask)


# speedup vs baseline: 1.5674x; 1.5674x over previous
import functools

import jax
import jax.numpy as jnp
from jax import lax
from jax.experimental import pallas as pl
from jax.experimental.pallas import tpu as pltpu
from jax.experimental.pallas import tpu_sc as plsc

N_CLASS = 1000000
DROPOUT_PROB = 0.1
ROW_SLOT = 128


def _drop_mask(n: int):
    u = jax.random.uniform(jax.random.key(1234), (n,))
    return (u < DROPOUT_PROB).astype(jnp.int32)


def _format_kernel(in_ref, out_ref):
    hidden = in_ref.shape[0]
    blk = in_ref.shape[1]
    eye = jnp.eye(hidden, ROW_SLOT, dtype=jnp.float32)
    out_ref[...] = lax.dot_general(
        in_ref[...], eye, (((0,), (0,)), ((), ())),
        preferred_element_type=jnp.float32,
    )


@functools.lru_cache
def _build_format(hidden: int, n_rows: int, blk: int):
    return pl.pallas_call(
        _format_kernel,
        grid=(pl.cdiv(n_rows, blk),),
        in_specs=[pl.BlockSpec((hidden, blk), lambda i: (0, i))],
        out_specs=pl.BlockSpec((blk, ROW_SLOT), lambda i: (i, 0)),
        out_shape=jax.ShapeDtypeStruct((n_rows, ROW_SLOT), jnp.float32),
    )


@functools.lru_cache
def _build_gather(batch: int, hidden: int, n_rows: int):
    info = plsc.get_sparse_core_info()
    nc, ns, lanes = info.num_cores, info.num_subcores, info.num_lanes
    nw = nc * ns
    assert batch % (8 * nw) == 0 and hidden % lanes == 0
    b_per_w = batch // nw
    mesh = plsc.VectorSubcoreMesh(core_axis_name="c", subcore_axis_name="s")

    @functools.partial(
        pl.kernel,
        mesh=mesh,
        out_type=jax.ShapeDtypeStruct((batch, ROW_SLOT), jnp.float32),
        compiler_params=pltpu.CompilerParams(use_tc_tiling_on_sc=True),
        scratch_types=[
            pltpu.VMEM((b_per_w,), jnp.int32),
            pltpu.VMEM((b_per_w,), jnp.int32),
            pltpu.VMEM((b_per_w,), jnp.int32),
            pltpu.VMEM((b_per_w, ROW_SLOT), jnp.float32),
            pltpu.SemaphoreType.DMA,
        ],
    )
    def emb(tab_hbm, labels_hbm, mask_hbm, out_hbm, lab_v, msk_v, idx_v, rows_v, sem):
        wid = lax.axis_index("s") * nc + lax.axis_index("c")
        base = wid * b_per_w
        pltpu.sync_copy(labels_hbm.at[pl.ds(base, b_per_w)], lab_v)
        pltpu.sync_copy(mask_hbm.at[pl.ds(base, b_per_w)], msk_v)
        for i in range(b_per_w // lanes):
            s = pl.ds(i * lanes, lanes)
            idx_v[s] = jnp.where(msk_v[s] != 0, N_CLASS, lab_v[s])
        pltpu.async_copy(tab_hbm.at[idx_v], rows_v, sem).wait()
        pltpu.sync_copy(rows_v, out_hbm.at[pl.ds(base, b_per_w)])

    return emb


def kernel(labels, table):
    batch = labels.shape[0]
    n_rows, hidden = table.shape
    mask = _drop_mask(batch)
    fmt = _build_format(hidden, n_rows, 2048)(table.T)
    emb = _build_gather(batch, hidden, n_rows)
    out = emb(fmt, labels.astype(jnp.int32), mask)
    return out[:, :hidden]
